# baseline (speedup 1.0000x reference)
import numpy as np
import jax
import jax.numpy as jnp
from jax import lax
from jax.experimental import pallas as pl
from jax.experimental.pallas import tpu as pltpu

_B, _D, _H, _W = 32, 64, 32, 32
_K = 1024
_N = _B * _H * _W
_R = 512
_GRID = _N // _R
_TPB = _H * _W // _R

_ROT = ((13, 15, 26, 6), (17, 29, 16, 24))
_KS = (np.uint32(0), np.uint32(42), np.uint32(42 ^ 0x1BD11BDA))


def _threefry_bits(x1_init):
    x1 = x1_init
    x0 = x1
    first = True
    for i in range(5):
        for r in _ROT[i % 2]:
            if first:
                first = False
            else:
                x0 = x0 + x1
            x1 = lax.shift_left(x1, np.uint32(r)) | lax.shift_right_logical(
                x1, np.uint32(32 - r))
            x1 = x1 ^ x0
        x0 = x0 + _KS[(i + 1) % 3]
        x1 = x1 + _KS[(i + 2) % 3] + np.uint32(i + 1)
    return x0 ^ x1


def _body(za_ref, zb_ref, emb_ref, kappa_ref, zq_ref, idx_ref, reg_ref,
          cnt_ref, emb_n_ref, score_ref):
    t = pl.program_id(0)

    @pl.when(t == 0)
    def _():
        row = lax.broadcasted_iota(jnp.uint32, (_R, _K), 0)
        col = lax.broadcasted_iota(jnp.uint32, (_R, _K), 1)
        cnt_ref[...] = row * np.uint32(_K) + col + _KS[1]
        emb = emb_ref[...]
        norm = jnp.sqrt(jnp.sum(emb * emb, axis=1, keepdims=True))
        emb_n_ref[...] = emb / jnp.maximum(norm, np.float32(1e-12))

    emb_n = emb_n_ref[...]
    kappa = kappa_ref[0, 0]

    t_a = jnp.minimum(t, _GRID - 1)
    zda = za_ref[0]
    sims_a = lax.dot_general(
        zda, emb_n, (((0,), (1,)), ((), ())),
        preferred_element_type=jnp.float32)
    base = lax.convert_element_type(t_a * (_R * _K), jnp.uint32)
    bits = _threefry_bits(cnt_ref[...] + base)
    u = lax.bitcast_convert_type(
        lax.shift_right_logical(bits, np.uint32(9)) | np.uint32(0x3F800000),
        jnp.float32) - np.float32(1.0)
    g = -jnp.log(-jnp.log(u))
    score_ref[t % 2] = g + kappa * sims_a

    score = score_ref[(t + 1) % 2]
    idx = jnp.argmax(score, axis=1)
    idx_ref[...] = idx.reshape(1, 1, _R)

    col = lax.broadcasted_iota(jnp.int32, (_R, _K), 1)
    onehot = jnp.where(col == idx[:, None], np.float32(1.0), np.float32(0.0))
    zq = lax.dot_general(
        onehot, emb_n, (((1,), (0,)), ((), ())),
        preferred_element_type=jnp.float32)
    zq_ref[...] = zq

    chosen = jnp.sum(zb_ref[0].T * zq, axis=1)
    reg_ref[0, 0, 0] = jnp.sum(kappa * (np.float32(1.0) - chosen))


def _quantize(z3, emb_weight, kappa2):
    def prev(t):
        return jnp.maximum(t - 1, 0)

    return pl.pallas_call(
        _body,
        grid=(_GRID + 1,),
        in_specs=[
            pl.BlockSpec((1, _D, _R),
                         lambda t: (jnp.minimum(t, _GRID - 1) // _TPB, 0,
                                    jnp.minimum(t, _GRID - 1) % _TPB)),
            pl.BlockSpec((1, _D, _R),
                         lambda t: (prev(t) // _TPB, 0, prev(t) % _TPB)),
            pl.BlockSpec((_K, _D), lambda t: (0, 0)),
            pl.BlockSpec((1, 1), lambda t: (0, 0), memory_space=pltpu.SMEM),
        ],
        out_specs=[
            pl.BlockSpec((_R, _D), lambda t: (prev(t), 0)),
            pl.BlockSpec((1, 1, _R), lambda t: (prev(t), 0, 0)),
            pl.BlockSpec((1, 1, 1), lambda t: (prev(t), 0, 0),
                         memory_space=pltpu.SMEM),
        ],
        out_shape=[
            jax.ShapeDtypeStruct((_N, _D), jnp.float32),
            jax.ShapeDtypeStruct((_GRID, 1, _R), jnp.int32),
            jax.ShapeDtypeStruct((_GRID, 1, 1), jnp.float32),
        ],
        scratch_shapes=[
            pltpu.VMEM((_R, _K), jnp.uint32),
            pltpu.VMEM((_K, _D), jnp.float32),
            pltpu.VMEM((2, _R, _K), jnp.float32),
        ],
    )(z3, z3, emb_weight, kappa2)


def kernel(z_e, emb_weight, kappa_phi):
    B, D, H, W = z_e.shape
    z3 = z_e.reshape(B, D, H * W)
    kappa2 = jnp.reshape(kappa_phi, (1, 1)).astype(jnp.float32)
    zq_flat, idx, reg_parts = _quantize(z3, emb_weight, kappa2)
    z_q = jnp.transpose(zq_flat.reshape(B, H, W, D), (0, 3, 1, 2))
    reg = (jnp.sum(reg_parts) / np.float32(_N)).astype(jnp.float32)
    indices = idx.reshape(B, H, W)
    return (z_q, reg, indices)


# speedup vs baseline: 1.0035x; 1.0035x over previous
import numpy as np
import jax
import jax.numpy as jnp
from jax import lax
from jax.experimental import pallas as pl
from jax.experimental.pallas import tpu as pltpu

_B, _D, _H, _W = 32, 64, 32, 32
_K = 1024
_N = _B * _H * _W
_R = 256
_GRID = _N // _R
_TPB = _H * _W // _R

_ROT = ((13, 15, 26, 6), (17, 29, 16, 24))
_KS = (np.uint32(0), np.uint32(42), np.uint32(42 ^ 0x1BD11BDA))


def _threefry_bits(x1_init):
    x1 = x1_init
    x0 = x1
    first = True
    for i in range(5):
        for r in _ROT[i % 2]:
            if first:
                first = False
            else:
                x0 = x0 + x1
            x1 = lax.shift_left(x1, np.uint32(r)) | lax.shift_right_logical(
                x1, np.uint32(32 - r))
            x1 = x1 ^ x0
        x0 = x0 + _KS[(i + 1) % 3]
        x1 = x1 + _KS[(i + 2) % 3] + np.uint32(i + 1)
    return x0 ^ x1


def _body(z_ref, emb_ref, kappa_ref, zq_ref, idx_ref, reg_ref,
          cnt_ref, emb_n_ref):
    t = pl.program_id(0)

    @pl.when(t == 0)
    def _():
        row = lax.broadcasted_iota(jnp.uint32, (_R, _K), 0)
        col = lax.broadcasted_iota(jnp.uint32, (_R, _K), 1)
        cnt_ref[...] = row * np.uint32(_K) + col + _KS[1]
        emb = emb_ref[...]
        norm = jnp.sqrt(jnp.sum(emb * emb, axis=1, keepdims=True))
        emb_n_ref[...] = emb / jnp.maximum(norm, np.float32(1e-12))

    emb_n = emb_n_ref[...]
    zd = z_ref[0]
    sims = lax.dot_general(
        zd, emb_n, (((0,), (1,)), ((), ())),
        preferred_element_type=jnp.float32)

    kappa = kappa_ref[0, 0]
    logits = kappa * sims

    base = lax.convert_element_type(t * (_R * _K), jnp.uint32)
    bits = _threefry_bits(cnt_ref[...] + base)

    u = lax.bitcast_convert_type(
        lax.shift_right_logical(bits, np.uint32(9)) | np.uint32(0x3F800000),
        jnp.float32) - np.float32(1.0)
    g = -jnp.log(-jnp.log(u))
    score = g + logits

    idx = jnp.argmax(score, axis=1)
    idx_ref[...] = idx.reshape(1, 1, _R)

    col = lax.broadcasted_iota(jnp.int32, (_R, _K), 1)
    onehot = jnp.where(col == idx[:, None], np.float32(1.0), np.float32(0.0))
    zq = lax.dot_general(
        onehot, emb_n, (((1,), (0,)), ((), ())),
        preferred_element_type=jnp.float32)
    zq_ref[...] = zq

    chosen = jnp.sum(zd.T * zq, axis=1)
    reg_ref[0, 0, 0] = jnp.sum(kappa * (np.float32(1.0) - chosen))


def _quantize(z3, emb_weight, kappa2):
    return pl.pallas_call(
        _body,
        grid=(_GRID,),
        in_specs=[
            pl.BlockSpec((1, _D, _R), lambda t: (t // _TPB, 0, t % _TPB)),
            pl.BlockSpec((_K, _D), lambda t: (0, 0)),
            pl.BlockSpec((1, 1), lambda t: (0, 0), memory_space=pltpu.SMEM),
        ],
        out_specs=[
            pl.BlockSpec((_R, _D), lambda t: (t, 0)),
            pl.BlockSpec((1, 1, _R), lambda t: (t, 0, 0)),
            pl.BlockSpec((1, 1, 1), lambda t: (t, 0, 0),
                         memory_space=pltpu.SMEM),
        ],
        out_shape=[
            jax.ShapeDtypeStruct((_N, _D), jnp.float32),
            jax.ShapeDtypeStruct((_GRID, 1, _R), jnp.int32),
            jax.ShapeDtypeStruct((_GRID, 1, 1), jnp.float32),
        ],
        scratch_shapes=[
            pltpu.VMEM((_R, _K), jnp.uint32),
            pltpu.VMEM((_K, _D), jnp.float32),
        ],
    )(z3, emb_weight, kappa2)


def kernel(z_e, emb_weight, kappa_phi):
    B, D, H, W = z_e.shape
    z3 = z_e.reshape(B, D, H * W)
    kappa2 = jnp.reshape(kappa_phi, (1, 1)).astype(jnp.float32)
    zq_flat, idx, reg_parts = _quantize(z3, emb_weight, kappa2)
    z_q = jnp.transpose(zq_flat.reshape(B, H, W, D), (0, 3, 1, 2))
    reg = (jnp.sum(reg_parts) / np.float32(_N)).astype(jnp.float32)
    indices = idx.reshape(B, H, W)
    return (z_q, reg, indices)


# speedup vs baseline: 1.0556x; 1.0519x over previous
import numpy as np
import jax
import jax.numpy as jnp
from jax import lax
from jax.experimental import pallas as pl
from jax.experimental.pallas import tpu as pltpu

_B, _D, _H, _W = 32, 64, 32, 32
_K = 1024
_N = _B * _H * _W
_R = 512
_GRID = _N // _R
_TPB = _H * _W // _R

_ROT = ((13, 15, 26, 6), (17, 29, 16, 24))
_KS = (np.uint32(0), np.uint32(42), np.uint32(42 ^ 0x1BD11BDA))


def _threefry_bits(x1_init):
    x1 = x1_init
    x0 = x1
    first = True
    for i in range(5):
        for r in _ROT[i % 2]:
            if first:
                first = False
            else:
                x0 = x0 + x1
            x1 = lax.shift_left(x1, np.uint32(r)) | lax.shift_right_logical(
                x1, np.uint32(32 - r))
            x1 = x1 ^ x0
        x0 = x0 + _KS[(i + 1) % 3]
        x1 = x1 + _KS[(i + 2) % 3] + np.uint32(i + 1)
    return x0 ^ x1


def _body(z_ref, emb_ref, kappa_ref, zq_ref, idx_ref, reg_ref,
          cnt_ref, emb_n_ref):
    t = pl.program_id(0)

    @pl.when(t == 0)
    def _():
        row = lax.broadcasted_iota(jnp.uint32, (_R, _K), 0)
        col = lax.broadcasted_iota(jnp.uint32, (_R, _K), 1)
        cnt_ref[...] = row * np.uint32(_K) + col + _KS[1]
        emb = emb_ref[...]
        norm = jnp.sqrt(jnp.sum(emb * emb, axis=1, keepdims=True))
        emb_n_ref[...] = emb / jnp.maximum(norm, np.float32(1e-12))

    emb_n = emb_n_ref[...]
    zd = z_ref[0]
    sims = lax.dot_general(
        zd, emb_n, (((0,), (1,)), ((), ())),
        preferred_element_type=jnp.float32)

    kappa = kappa_ref[0, 0]
    logits = kappa * sims

    base = lax.convert_element_type(t * (_R * _K), jnp.uint32)
    bits = _threefry_bits(cnt_ref[...] + base)

    u = lax.bitcast_convert_type(
        lax.shift_right_logical(bits, np.uint32(9)) | np.uint32(0x3F800000),
        jnp.float32) - np.float32(1.0)
    g = -jnp.log(-jnp.log(u))
    score = g + logits

    idx = jnp.argmax(score, axis=1)
    idx_ref[...] = idx.reshape(1, 1, _R)

    col = lax.broadcasted_iota(jnp.int32, (_R, _K), 1)
    onehot = jnp.where(col == idx[:, None], np.float32(1.0), np.float32(0.0))
    zq = lax.dot_general(
        onehot, emb_n, (((1,), (0,)), ((), ())),
        preferred_element_type=jnp.float32)
    zq_ref[...] = zq

    chosen_total = jnp.sum(zd.T * zq)
    reg_ref[0, 0, 0] = kappa * (np.float32(_R) - chosen_total)


def _quantize(z3, emb_weight, kappa2):
    return pl.pallas_call(
        _body,
        grid=(_GRID,),
        in_specs=[
            pl.BlockSpec((1, _D, _R), lambda t: (t // _TPB, 0, t % _TPB)),
            pl.BlockSpec((_K, _D), lambda t: (0, 0)),
            pl.BlockSpec((1, 1), lambda t: (0, 0), memory_space=pltpu.SMEM),
        ],
        out_specs=[
            pl.BlockSpec((_R, _D), lambda t: (t, 0)),
            pl.BlockSpec((1, 1, _R), lambda t: (t, 0, 0)),
            pl.BlockSpec((1, 1, 1), lambda t: (t, 0, 0),
                         memory_space=pltpu.SMEM),
        ],
        out_shape=[
            jax.ShapeDtypeStruct((_N, _D), jnp.float32),
            jax.ShapeDtypeStruct((_GRID, 1, _R), jnp.int32),
            jax.ShapeDtypeStruct((_GRID, 1, 1), jnp.float32),
        ],
        scratch_shapes=[
            pltpu.VMEM((_R, _K), jnp.uint32),
            pltpu.VMEM((_K, _D), jnp.float32),
        ],
    )(z3, emb_weight, kappa2)


def kernel(z_e, emb_weight, kappa_phi):
    B, D, H, W = z_e.shape
    z3 = z_e.reshape(B, D, H * W)
    kappa2 = jnp.reshape(kappa_phi, (1, 1)).astype(jnp.float32)
    zq_flat, idx, reg_parts = _quantize(z3, emb_weight, kappa2)
    z_q = jnp.transpose(zq_flat.reshape(B, H, W, D), (0, 3, 1, 2))
    reg = (jnp.sum(reg_parts) / np.float32(_N)).astype(jnp.float32)
    indices = idx.reshape(B, H, W)
    return (z_q, reg, indices)
